# Initial kernel scaffold; baseline (speedup 1.0000x reference)
import jax
import jax.numpy as jnp
from jax.experimental import pallas as pl


def kernel(x_UE, x_AP, edge_attr, edge_index, batch, params):
    raise NotImplementedError("write your pallas kernel here")


# speedup vs baseline: 5.9756x; 5.9756x over previous
import functools

import jax
import jax.numpy as jnp
import numpy as np
from jax import lax
from jax.experimental import pallas as pl
from jax.experimental.pallas import tpu as pltpu
from jax.experimental.pallas import tpu_sc as plsc

_N = 10000
_NP = 10112
_E = 320000
_NW = 32
_EPW = _E // _NW
_C = 2000
_NCH = _EPW // _C

_TWO_PI = float(2.0 * np.pi)
_INV_2PI = float(1.0 / (2.0 * np.pi))
_COS_C = (1.0, -1.0 / 2, 1.0 / 24, -1.0 / 720, 1.0 / 40320,
          -1.0 / 3628800, 1.0 / 479001600, -1.0 / 87178291200)


def _leaky(x):
    return jnp.where(x > 0, x, 0.01 * x)


def _node_body(xue_ref, xap_ref, Wnu1, bnu1, Wnu2, bnu2, Wna1, bna1,
               Wsu, bsu, Wdu, bdu, hue_ref, su_ref, du_ref):
    a1 = _leaky(jnp.dot(xue_ref[...], Wnu1[...],
                        preferred_element_type=jnp.float32) + bnu1[...])
    hue_ref[...] = jnp.dot(a1, Wnu2[...],
                           preferred_element_type=jnp.float32) + bnu2[...]
    du_ref[...] = jnp.dot(a1, Wdu[...],
                          preferred_element_type=jnp.float32) + bdu[...]
    a2 = _leaky(jnp.dot(xap_ref[...], Wna1[...],
                        preferred_element_type=jnp.float32) + bna1[...])
    su_ref[...] = jnp.dot(a2, Wsu[...],
                          preferred_element_type=jnp.float32) + bsu[...]


def _edge_body(ea_ref, We1, be1, We23, be3, ec_ref):
    h = _leaky(jnp.dot(ea_ref[...], We1[...],
                       preferred_element_type=jnp.float32) + be1[...])
    ec_ref[...] = jnp.dot(h, We23[...],
                          preferred_element_type=jnp.float32) + be3[...]


def _sc_body(su_hbm, du_hbm, src_hbm, dst_hbm, ec_hbm, out_hbm,
             su_t, du_t, acc, srcb, dstb, ecb):
    wid = lax.axis_index("s") * 2 + lax.axis_index("c")

    pltpu.sync_copy(su_hbm, su_t)
    pltpu.sync_copy(du_hbm, du_t)

    zero16 = jnp.zeros((16,), jnp.float32)

    def _zbody(i, carry):
        acc[pl.ds(i * 16, 16)] = zero16
        return carry

    lax.fori_loop(0, (_NP * 4) // 16, _zbody, 0)

    iota16 = lax.iota(jnp.int32, 16)
    eidx0 = iota16 * 3
    ones16 = jnp.full((16,), 1.0, jnp.float32)

    def _gbody(g, carry):
        rs = srcb[pl.ds(g * 16, 16)]
        rd = dstb[pl.ds(g * 16, 16)]
        rs3 = rs * 3
        rd3 = rd * 3
        ebase = g * 48 + eidx0
        for j in range(3):
            sj = plsc.load_gather(su_t, [rs3 + j])
            dj = plsc.load_gather(du_t, [rd3 + j])
            ej = plsc.load_gather(ecb, [ebase + j])
            x = ej + sj + dj
            k = x * _INV_2PI
            kf = (k + jnp.sign(k) * 0.5).astype(jnp.int32).astype(jnp.float32)
            r = x - kf * _TWO_PI
            y = r * r
            pv = jnp.full((16,), _COS_C[7], jnp.float32)
            for c in (_COS_C[6], _COS_C[5], _COS_C[4], _COS_C[3],
                      _COS_C[2], _COS_C[1], _COS_C[0]):
                pv = pv * y + c
            plsc.addupdate_scatter(acc, [rd + j * _NP], pv)
        plsc.addupdate_scatter(acc, [rd + 3 * _NP], ones16)
        return carry

    for ch in range(_NCH):
        base = wid * _EPW + ch * _C
        pltpu.sync_copy(src_hbm.at[pl.ds(base, _C)], srcb)
        pltpu.sync_copy(dst_hbm.at[pl.ds(base, _C)], dstb)
        pltpu.sync_copy(ec_hbm.at[pl.ds(base * 3, _C * 3)], ecb)
        lax.fori_loop(0, _C // 16, _gbody, 0)

    pltpu.sync_copy(acc, out_hbm.at[pl.ds(wid * (4 * _NP), 4 * _NP)])


def _sc_edges(su, du, src, dst, ec):
    run = functools.partial(
        pl.kernel,
        out_type=jax.ShapeDtypeStruct((_NW * 4 * _NP,), jnp.float32),
        mesh=plsc.VectorSubcoreMesh(core_axis_name="c", subcore_axis_name="s",
                                    num_cores=2, num_subcores=16),
        compiler_params=pltpu.CompilerParams(needs_layout_passes=False),
        scratch_types=[
            pltpu.VMEM((_N * 3,), jnp.float32),
            pltpu.VMEM((_N * 3,), jnp.float32),
            pltpu.VMEM((_NP * 4,), jnp.float32),
            pltpu.VMEM((_C,), jnp.int32),
            pltpu.VMEM((_C,), jnp.int32),
            pltpu.VMEM((_C * 3,), jnp.float32),
        ],
    )(_sc_body)
    return run(su, du, src, dst, ec)


def _post_body(parts_ref, hueT_ref,
               Wu1aT, Wu1bT, bu1T, Wu2T, bu2T,
               WtT, btT, Wf2T, bf2T, Wf3T, bf3T, out_ref):
    parts = parts_ref[...]
    m0 = jnp.sum(parts[:, 0:_N], axis=0, keepdims=True)
    m1 = jnp.sum(parts[:, _NP:_NP + _N], axis=0, keepdims=True)
    m2 = jnp.sum(parts[:, 2 * _NP:2 * _NP + _N], axis=0, keepdims=True)
    deg = jnp.sum(parts[:, 3 * _NP:3 * _NP + _N], axis=0, keepdims=True)
    inv = 1.0 / jnp.maximum(deg, 1.0)
    a0 = m0 * inv
    a1 = m1 * inv
    a2 = m2 * inv
    hueT = hueT_ref[...]
    h0 = hueT[0:1, :]
    h1 = hueT[1:2, :]
    u = _leaky(h0 * Wu1aT[:, 0:1] + h1 * Wu1aT[:, 1:2]
               + a0 * Wu1bT[:, 0:1] + a1 * Wu1bT[:, 1:2]
               + a2 * Wu1bT[:, 2:3] + bu1T[...])
    upd = jnp.dot(Wu2T[...], u,
                  preferred_element_type=jnp.float32) + bu2T[...]
    mask = (deg > 0).astype(jnp.float32)
    h = hueT + upd * mask
    diff = (h[0:1, :] - h[1:2, :]) * 0.5
    t = diff * lax.rsqrt(diff * diff + 1e-5)
    f = _leaky(t * WtT[...] + btT[...])
    f = _leaky(jnp.dot(Wf2T[...], f,
                       preferred_element_type=jnp.float32) + bf2T[...])
    o = jnp.dot(Wf3T[...], f,
                preferred_element_type=jnp.float32) + bf3T[...]
    out_ref[...] = jax.nn.sigmoid(o)


def kernel(x_UE, x_AP, edge_attr, edge_index, batch, params):
    p = params
    f32 = jnp.float32

    qc_e = p["qc_W"][0:2]
    qc_su = p["qc_W"][2:4]
    qc_du = p["qc_W"][4:6]
    Wsu = p["W_na2"] @ qc_su
    bsu = (p["b_na2"] @ qc_su).reshape(1, 3)
    Wdu = p["W_nu2"] @ qc_du
    bdu = (p["b_nu2"] @ qc_du).reshape(1, 3)
    We23 = p["W_e2"] @ qc_e
    be3 = (p["b_e2"] @ qc_e + p["qc_b"]).reshape(1, 3)
    g0 = p["ln_g"][0]
    g1 = p["ln_g"][1]
    b0 = p["ln_b"][0]
    b1 = p["ln_b"][1]
    Wt = (g0 * p["W_f1"][0:1, :] - g1 * p["W_f1"][1:2, :])
    bt = (b0 * p["W_f1"][0:1, :] + b1 * p["W_f1"][1:2, :]
          + p["b_f1"].reshape(1, -1))

    hue, su, du = pl.pallas_call(
        _node_body,
        out_shape=(
            jax.ShapeDtypeStruct((_N, 2), f32),
            jax.ShapeDtypeStruct((_N, 3), f32),
            jax.ShapeDtypeStruct((_N, 3), f32),
        ),
    )(x_UE, x_AP,
      p["W_nu1"], p["b_nu1"].reshape(1, -1), p["W_nu2"],
      p["b_nu2"].reshape(1, -1), p["W_na1"], p["b_na1"].reshape(1, -1),
      Wsu, bsu, Wdu, bdu)

    be = 4000
    ec = pl.pallas_call(
        _edge_body,
        grid=(_E // be,),
        in_specs=[
            pl.BlockSpec((be, 16), lambda i: (i, 0)),
            pl.BlockSpec((16, 64), lambda i: (0, 0)),
            pl.BlockSpec((1, 64), lambda i: (0, 0)),
            pl.BlockSpec((64, 3), lambda i: (0, 0)),
            pl.BlockSpec((1, 3), lambda i: (0, 0)),
        ],
        out_specs=pl.BlockSpec((be, 3), lambda i: (i, 0)),
        out_shape=jax.ShapeDtypeStruct((_E, 3), f32),
    )(edge_attr, p["W_e1"], p["b_e1"].reshape(1, -1), We23, be3)

    src = edge_index[0].astype(jnp.int32)
    dst = edge_index[1].astype(jnp.int32)
    parts = _sc_edges(su.reshape(-1), du.reshape(-1), src, dst,
                      ec.reshape(-1))

    outT = pl.pallas_call(
        _post_body,
        out_shape=jax.ShapeDtypeStruct((2, _N), f32),
    )(parts.reshape(_NW, 4 * _NP), hue.T,
      p["W_u1"][0:2].T, p["W_u1"][2:5].T, p["b_u1"].reshape(-1, 1),
      p["W_u2"].T, p["b_u2"].reshape(-1, 1),
      Wt.reshape(-1, 1), bt.reshape(-1, 1),
      p["W_f2"].T, p["b_f2"].reshape(-1, 1),
      p["W_f3"].T, p["b_f3"].reshape(-1, 1))
    return outT.T


# speedup vs baseline: 7.0673x; 1.1827x over previous
import functools

import jax
import jax.numpy as jnp
import numpy as np
from jax import lax
from jax.experimental import pallas as pl
from jax.experimental.pallas import tpu as pltpu
from jax.experimental.pallas import tpu_sc as plsc

_N = 10000
_NP = 10112
_E = 320000
_NW = 32
_EPW = _E // _NW
_C = 2000
_NCH = _EPW // _C

_TWO_PI = float(2.0 * np.pi)
_INV_2PI = float(1.0 / (2.0 * np.pi))
_COS_C = (1.0, -1.0 / 2, 1.0 / 24, -1.0 / 720, 1.0 / 40320,
          -1.0 / 3628800, 1.0 / 479001600, -1.0 / 87178291200)


def _leaky(x):
    return jnp.where(x > 0, x, 0.01 * x)


def _node_body(xue_ref, xap_ref, Wnu1, bnu1, Wnu2, bnu2, Wna1, bna1,
               Wsu, bsu, Wdu, bdu, hue_ref, su_ref, du_ref):
    a1 = _leaky(jnp.dot(xue_ref[...], Wnu1[...],
                        preferred_element_type=jnp.float32) + bnu1[...])
    hue_ref[...] = jnp.dot(a1, Wnu2[...],
                           preferred_element_type=jnp.float32) + bnu2[...]
    du_ref[...] = jnp.dot(a1, Wdu[...],
                          preferred_element_type=jnp.float32) + bdu[...]
    a2 = _leaky(jnp.dot(xap_ref[...], Wna1[...],
                        preferred_element_type=jnp.float32) + bna1[...])
    su_ref[...] = jnp.dot(a2, Wsu[...],
                          preferred_element_type=jnp.float32) + bsu[...]


def _edge_body(ea_ref, We1, be1, We23, be3, ec_ref):
    h = _leaky(jnp.dot(ea_ref[...], We1[...],
                       preferred_element_type=jnp.float32) + be1[...])
    ec_ref[...] = jnp.dot(h, We23[...],
                          preferred_element_type=jnp.float32) + be3[...]


_RND = 12582912.0


def _sc_body(su_hbm, du_hbm, src_hbm, dst_hbm, ec_hbm, out_hbm,
             su_t, du_t, acc, srcb, dstb, ecb, sem_t, sem0, sem1):
    wid = lax.axis_index("s") * 2 + lax.axis_index("c")

    h_su = pltpu.async_copy(su_hbm, su_t, sem_t)
    h_du = pltpu.async_copy(du_hbm, du_t, sem_t)

    sems = (sem0, sem1)

    def _start_chunk(ch):
        b = ch % 2
        base = wid * _EPW + ch * _C
        hs = pltpu.async_copy(src_hbm.at[pl.ds(base, _C)],
                              srcb.at[pl.ds(b * _C, _C)], sems[b])
        hd = pltpu.async_copy(dst_hbm.at[pl.ds(base, _C)],
                              dstb.at[pl.ds(b * _C, _C)], sems[b])
        he = pltpu.async_copy(ec_hbm.at[pl.ds(base * 3, _C * 3)],
                              ecb.at[pl.ds(b * _C * 3, _C * 3)], sems[b])
        return (hs, hd, he)

    pend = _start_chunk(0)

    zero16 = jnp.zeros((16,), jnp.float32)

    @plsc.parallel_loop(0, (_NP * 4) // 16, unroll=8)
    def _zbody(i):
        acc[pl.ds(i * 16, 16)] = zero16

    h_su.wait()
    h_du.wait()

    iota16 = lax.iota(jnp.int32, 16)
    eidx0 = iota16 * 3
    ones16 = jnp.full((16,), 1.0, jnp.float32)

    for ch in range(_NCH):
        b = ch % 2
        for h in pend:
            h.wait()
        if ch + 1 < _NCH:
            pend = _start_chunk(ch + 1)
        soff = b * _C
        eoff = b * _C * 3

        @plsc.parallel_loop(0, _C // 16, unroll=4)
        def _gbody(g):
            rs = srcb[pl.ds(soff + g * 16, 16)]
            rd = dstb[pl.ds(soff + g * 16, 16)]
            rs3 = rs * 3
            rd3 = rd * 3
            ebase = eoff + g * 48 + eidx0
            for j in range(3):
                sj = plsc.load_gather(su_t, [rs3 + j])
                dj = plsc.load_gather(du_t, [rd3 + j])
                ej = plsc.load_gather(ecb, [ebase + j])
                x = ej + sj + dj
                kf = (x * _INV_2PI + _RND) - _RND
                r = x - kf * _TWO_PI
                y = r * r
                pv = jnp.full((16,), _COS_C[7], jnp.float32)
                for c in (_COS_C[6], _COS_C[5], _COS_C[4], _COS_C[3],
                          _COS_C[2], _COS_C[1], _COS_C[0]):
                    pv = pv * y + c
                plsc.addupdate_scatter(acc, [rd + j * _NP], pv)
            plsc.addupdate_scatter(acc, [rd + 3 * _NP], ones16)

    pltpu.sync_copy(acc, out_hbm.at[pl.ds(wid * (4 * _NP), 4 * _NP)])


def _sc_edges(su, du, src, dst, ec):
    run = functools.partial(
        pl.kernel,
        out_type=jax.ShapeDtypeStruct((_NW * 4 * _NP,), jnp.float32),
        mesh=plsc.VectorSubcoreMesh(core_axis_name="c", subcore_axis_name="s",
                                    num_cores=2, num_subcores=16),
        compiler_params=pltpu.CompilerParams(needs_layout_passes=False),
        scratch_types=[
            pltpu.VMEM((_N * 3,), jnp.float32),
            pltpu.VMEM((_N * 3,), jnp.float32),
            pltpu.VMEM((_NP * 4,), jnp.float32),
            pltpu.VMEM((2 * _C,), jnp.int32),
            pltpu.VMEM((2 * _C,), jnp.int32),
            pltpu.VMEM((2 * _C * 3,), jnp.float32),
            pltpu.SemaphoreType.DMA,
            pltpu.SemaphoreType.DMA,
            pltpu.SemaphoreType.DMA,
        ],
    )(_sc_body)
    return run(su, du, src, dst, ec)


def _post_body(parts_ref, hueT_ref,
               Wu1aT, Wu1bT, bu1T, Wu2T, bu2T,
               WtT, btT, Wf2T, bf2T, Wf3T, bf3T, out_ref):
    parts = parts_ref[...]
    m0 = jnp.sum(parts[:, 0:_N], axis=0, keepdims=True)
    m1 = jnp.sum(parts[:, _NP:_NP + _N], axis=0, keepdims=True)
    m2 = jnp.sum(parts[:, 2 * _NP:2 * _NP + _N], axis=0, keepdims=True)
    deg = jnp.sum(parts[:, 3 * _NP:3 * _NP + _N], axis=0, keepdims=True)
    inv = 1.0 / jnp.maximum(deg, 1.0)
    a0 = m0 * inv
    a1 = m1 * inv
    a2 = m2 * inv
    hueT = hueT_ref[...]
    h0 = hueT[0:1, :]
    h1 = hueT[1:2, :]
    u = _leaky(h0 * Wu1aT[:, 0:1] + h1 * Wu1aT[:, 1:2]
               + a0 * Wu1bT[:, 0:1] + a1 * Wu1bT[:, 1:2]
               + a2 * Wu1bT[:, 2:3] + bu1T[...])
    upd = jnp.dot(Wu2T[...], u,
                  preferred_element_type=jnp.float32) + bu2T[...]
    mask = (deg > 0).astype(jnp.float32)
    h = hueT + upd * mask
    diff = (h[0:1, :] - h[1:2, :]) * 0.5
    t = diff * lax.rsqrt(diff * diff + 1e-5)
    f = _leaky(t * WtT[...] + btT[...])
    f = _leaky(jnp.dot(Wf2T[...], f,
                       preferred_element_type=jnp.float32) + bf2T[...])
    o = jnp.dot(Wf3T[...], f,
                preferred_element_type=jnp.float32) + bf3T[...]
    out_ref[...] = jax.nn.sigmoid(o)


def kernel(x_UE, x_AP, edge_attr, edge_index, batch, params):
    p = params
    f32 = jnp.float32

    qc_e = p["qc_W"][0:2]
    qc_su = p["qc_W"][2:4]
    qc_du = p["qc_W"][4:6]
    Wsu = p["W_na2"] @ qc_su
    bsu = (p["b_na2"] @ qc_su).reshape(1, 3)
    Wdu = p["W_nu2"] @ qc_du
    bdu = (p["b_nu2"] @ qc_du).reshape(1, 3)
    We23 = p["W_e2"] @ qc_e
    be3 = (p["b_e2"] @ qc_e + p["qc_b"]).reshape(1, 3)
    g0 = p["ln_g"][0]
    g1 = p["ln_g"][1]
    b0 = p["ln_b"][0]
    b1 = p["ln_b"][1]
    Wt = (g0 * p["W_f1"][0:1, :] - g1 * p["W_f1"][1:2, :])
    bt = (b0 * p["W_f1"][0:1, :] + b1 * p["W_f1"][1:2, :]
          + p["b_f1"].reshape(1, -1))

    hue, su, du = pl.pallas_call(
        _node_body,
        out_shape=(
            jax.ShapeDtypeStruct((_N, 2), f32),
            jax.ShapeDtypeStruct((_N, 3), f32),
            jax.ShapeDtypeStruct((_N, 3), f32),
        ),
    )(x_UE, x_AP,
      p["W_nu1"], p["b_nu1"].reshape(1, -1), p["W_nu2"],
      p["b_nu2"].reshape(1, -1), p["W_na1"], p["b_na1"].reshape(1, -1),
      Wsu, bsu, Wdu, bdu)

    be = 4000
    ec = pl.pallas_call(
        _edge_body,
        grid=(_E // be,),
        in_specs=[
            pl.BlockSpec((be, 16), lambda i: (i, 0)),
            pl.BlockSpec((16, 64), lambda i: (0, 0)),
            pl.BlockSpec((1, 64), lambda i: (0, 0)),
            pl.BlockSpec((64, 3), lambda i: (0, 0)),
            pl.BlockSpec((1, 3), lambda i: (0, 0)),
        ],
        out_specs=pl.BlockSpec((be, 3), lambda i: (i, 0)),
        out_shape=jax.ShapeDtypeStruct((_E, 3), f32),
    )(edge_attr, p["W_e1"], p["b_e1"].reshape(1, -1), We23, be3)

    src = edge_index[0].astype(jnp.int32)
    dst = edge_index[1].astype(jnp.int32)
    parts = _sc_edges(su.reshape(-1), du.reshape(-1), src, dst,
                      ec.reshape(-1))

    outT = pl.pallas_call(
        _post_body,
        out_shape=jax.ShapeDtypeStruct((2, _N), f32),
    )(parts.reshape(_NW, 4 * _NP), hue.T,
      p["W_u1"][0:2].T, p["W_u1"][2:5].T, p["b_u1"].reshape(-1, 1),
      p["W_u2"].T, p["b_u2"].reshape(-1, 1),
      Wt.reshape(-1, 1), bt.reshape(-1, 1),
      p["W_f2"].T, p["b_f2"].reshape(-1, 1),
      p["W_f3"].T, p["b_f3"].reshape(-1, 1))
    return outT.T


# speedup vs baseline: 24.1778x; 3.4211x over previous
import functools

import jax
import jax.numpy as jnp
import numpy as np
from jax import lax
from jax.experimental import pallas as pl
from jax.experimental.pallas import tpu as pltpu
from jax.experimental.pallas import tpu_sc as plsc

_N = 10000
_NP = 10112
_E = 320000
_NW = 32
_EPW = _E // _NW
_C = 2000
_NCH = _EPW // _C

_TWO_PI = float(2.0 * np.pi)
_INV_2PI = float(1.0 / (2.0 * np.pi))
_COS_C = (1.0, -1.0 / 2, 1.0 / 24, -1.0 / 720, 1.0 / 40320,
          -1.0 / 3628800, 1.0 / 479001600, -1.0 / 87178291200)
_RND = 12582912.0


def _leaky(x):
    return jnp.where(x > 0, x, 0.01 * x)


def _node_body(xueT_ref, xapT_ref, Wnu1T, bnu1T, Wnu2T, bnu2T, Wna1T, bna1T,
               WsuT, bsuT, WduT, bduT, hueT_ref, suT_ref, duT_ref):
    a1 = _leaky(jnp.dot(Wnu1T[...], xueT_ref[...],
                        preferred_element_type=jnp.float32) + bnu1T[...])
    hueT_ref[...] = jnp.dot(Wnu2T[...], a1,
                            preferred_element_type=jnp.float32) + bnu2T[...]
    duT_ref[...] = jnp.dot(WduT[...], a1,
                           preferred_element_type=jnp.float32) + bduT[...]
    a2 = _leaky(jnp.dot(Wna1T[...], xapT_ref[...],
                        preferred_element_type=jnp.float32) + bna1T[...])
    suT_ref[...] = jnp.dot(WsuT[...], a2,
                           preferred_element_type=jnp.float32) + bsuT[...]


def _edge_body(eaT_ref, We1T, be1T, We23T, be3T, ecT_ref):
    h = _leaky(jnp.dot(We1T[...], eaT_ref[...],
                       preferred_element_type=jnp.float32) + be1T[...])
    ecT_ref[...] = jnp.dot(We23T[...], h,
                           preferred_element_type=jnp.float32) + be3T[...]


def _sc_body(su_hbm, du_hbm, src_hbm, dst_hbm, ec_hbm, out_hbm,
             su_t, du_t, acc, srcb, dstb, ecb, sem_t, sem0, sem1):
    wid = lax.axis_index("s") * 2 + lax.axis_index("c")

    h_su = pltpu.async_copy(su_hbm, su_t, sem_t)
    h_du = pltpu.async_copy(du_hbm, du_t, sem_t)

    sems = (sem0, sem1)

    def _start_chunk(ch):
        b = ch % 2
        base = wid * _EPW + ch * _C
        hs = pltpu.async_copy(src_hbm.at[pl.ds(base, _C)],
                              srcb.at[pl.ds(b * _C, _C)], sems[b])
        hd = pltpu.async_copy(dst_hbm.at[pl.ds(base, _C)],
                              dstb.at[pl.ds(b * _C, _C)], sems[b])
        he = tuple(
            pltpu.async_copy(ec_hbm.at[pl.ds(j * _E + base, _C)],
                             ecb.at[pl.ds((b * 3 + j) * _C, _C)], sems[b])
            for j in range(3))
        return (hs, hd) + he

    pend = _start_chunk(0)

    zero16 = jnp.zeros((16,), jnp.float32)

    @plsc.parallel_loop(0, (_NP * 4) // 16, unroll=8)
    def _zbody(i):
        acc[pl.ds(i * 16, 16)] = zero16

    h_su.wait()
    h_du.wait()

    ones16 = jnp.full((16,), 1.0, jnp.float32)

    for ch in range(_NCH):
        b = ch % 2
        for h in pend:
            h.wait()
        if ch + 1 < _NCH:
            pend = _start_chunk(ch + 1)
        soff = b * _C
        eoff = b * 3 * _C

        @plsc.parallel_loop(0, _C // 16, unroll=4)
        def _gbody(g):
            g16 = g * 16
            rs = srcb[pl.ds(soff + g16, 16)]
            rd = dstb[pl.ds(soff + g16, 16)]
            for j in range(3):
                sj = plsc.load_gather(su_t, [rs + j * _N])
                dj = plsc.load_gather(du_t, [rd + j * _N])
                ej = ecb[pl.ds(eoff + j * _C + g16, 16)]
                x = ej + sj + dj
                kf = (x * _INV_2PI + _RND) - _RND
                r = x - kf * _TWO_PI
                y = r * r
                pv = jnp.full((16,), _COS_C[7], jnp.float32)
                for c in (_COS_C[6], _COS_C[5], _COS_C[4], _COS_C[3],
                          _COS_C[2], _COS_C[1], _COS_C[0]):
                    pv = pv * y + c
                plsc.addupdate_scatter(acc, [rd + j * _NP], pv)
            plsc.addupdate_scatter(acc, [rd + 3 * _NP], ones16)

    pltpu.sync_copy(acc, out_hbm.at[pl.ds(wid * (4 * _NP), 4 * _NP)])


def _sc_edges(su, du, src, dst, ec):
    run = functools.partial(
        pl.kernel,
        out_type=jax.ShapeDtypeStruct((_NW * 4 * _NP,), jnp.float32),
        mesh=plsc.VectorSubcoreMesh(core_axis_name="c", subcore_axis_name="s",
                                    num_cores=2, num_subcores=16),
        compiler_params=pltpu.CompilerParams(needs_layout_passes=False),
        scratch_types=[
            pltpu.VMEM((_N * 3,), jnp.float32),
            pltpu.VMEM((_N * 3,), jnp.float32),
            pltpu.VMEM((_NP * 4,), jnp.float32),
            pltpu.VMEM((2 * _C,), jnp.int32),
            pltpu.VMEM((2 * _C,), jnp.int32),
            pltpu.VMEM((2 * 3 * _C,), jnp.float32),
            pltpu.SemaphoreType.DMA,
            pltpu.SemaphoreType.DMA,
            pltpu.SemaphoreType.DMA,
        ],
    )(_sc_body)
    return run(su, du, src, dst, ec)


def _post_body(parts_ref, hueT_ref,
               Wu1aT, Wu1bT, bu1T, Wu2T, bu2T,
               WtT, btT, Wf2T, bf2T, Wf3T, bf3T, out_ref):
    parts = parts_ref[...]
    m0 = jnp.sum(parts[:, 0:_N], axis=0, keepdims=True)
    m1 = jnp.sum(parts[:, _NP:_NP + _N], axis=0, keepdims=True)
    m2 = jnp.sum(parts[:, 2 * _NP:2 * _NP + _N], axis=0, keepdims=True)
    deg = jnp.sum(parts[:, 3 * _NP:3 * _NP + _N], axis=0, keepdims=True)
    inv = 1.0 / jnp.maximum(deg, 1.0)
    a0 = m0 * inv
    a1 = m1 * inv
    a2 = m2 * inv
    hueT = hueT_ref[...]
    h0 = hueT[0:1, :]
    h1 = hueT[1:2, :]
    u = _leaky(h0 * Wu1aT[:, 0:1] + h1 * Wu1aT[:, 1:2]
               + a0 * Wu1bT[:, 0:1] + a1 * Wu1bT[:, 1:2]
               + a2 * Wu1bT[:, 2:3] + bu1T[...])
    upd = jnp.dot(Wu2T[...], u,
                  preferred_element_type=jnp.float32) + bu2T[...]
    mask = (deg > 0).astype(jnp.float32)
    h = hueT + upd * mask
    diff = (h[0:1, :] - h[1:2, :]) * 0.5
    t = diff * lax.rsqrt(diff * diff + 1e-5)
    f = _leaky(t * WtT[...] + btT[...])
    f = _leaky(jnp.dot(Wf2T[...], f,
                       preferred_element_type=jnp.float32) + bf2T[...])
    o = jnp.dot(Wf3T[...], f,
                preferred_element_type=jnp.float32) + bf3T[...]
    out_ref[...] = jax.nn.sigmoid(o)


def kernel(x_UE, x_AP, edge_attr, edge_index, batch, params):
    p = params
    f32 = jnp.float32

    qc_e = p["qc_W"][0:2]
    qc_su = p["qc_W"][2:4]
    qc_du = p["qc_W"][4:6]
    WsuT = (p["W_na2"] @ qc_su).T
    bsuT = (p["b_na2"] @ qc_su).reshape(3, 1)
    WduT = (p["W_nu2"] @ qc_du).T
    bduT = (p["b_nu2"] @ qc_du).reshape(3, 1)
    We23T = (p["W_e2"] @ qc_e).T
    be3T = (p["b_e2"] @ qc_e + p["qc_b"]).reshape(3, 1)
    g0 = p["ln_g"][0]
    g1 = p["ln_g"][1]
    b0 = p["ln_b"][0]
    b1 = p["ln_b"][1]
    Wt = (g0 * p["W_f1"][0:1, :] - g1 * p["W_f1"][1:2, :])
    bt = (b0 * p["W_f1"][0:1, :] + b1 * p["W_f1"][1:2, :]
          + p["b_f1"].reshape(1, -1))

    hueT, suT, duT = pl.pallas_call(
        _node_body,
        out_shape=(
            jax.ShapeDtypeStruct((2, _N), f32),
            jax.ShapeDtypeStruct((3, _N), f32),
            jax.ShapeDtypeStruct((3, _N), f32),
        ),
    )(x_UE.T, x_AP.T,
      p["W_nu1"].T, p["b_nu1"].reshape(-1, 1), p["W_nu2"].T,
      p["b_nu2"].reshape(-1, 1), p["W_na1"].T, p["b_na1"].reshape(-1, 1),
      WsuT, bsuT, WduT, bduT)

    be = 16000
    ecT = pl.pallas_call(
        _edge_body,
        grid=(_E // be,),
        in_specs=[
            pl.BlockSpec((16, be), lambda i: (0, i)),
            pl.BlockSpec((64, 16), lambda i: (0, 0)),
            pl.BlockSpec((64, 1), lambda i: (0, 0)),
            pl.BlockSpec((3, 64), lambda i: (0, 0)),
            pl.BlockSpec((3, 1), lambda i: (0, 0)),
        ],
        out_specs=pl.BlockSpec((3, be), lambda i: (0, i)),
        out_shape=jax.ShapeDtypeStruct((3, _E), f32),
    )(edge_attr.T, p["W_e1"].T, p["b_e1"].reshape(-1, 1), We23T, be3T)

    src = edge_index[0].astype(jnp.int32)
    dst = edge_index[1].astype(jnp.int32)
    parts = _sc_edges(suT.reshape(-1), duT.reshape(-1), src, dst,
                      ecT.reshape(-1))

    outT = pl.pallas_call(
        _post_body,
        out_shape=jax.ShapeDtypeStruct((2, _N), f32),
    )(parts.reshape(_NW, 4 * _NP), hueT,
      p["W_u1"][0:2].T, p["W_u1"][2:5].T, p["b_u1"].reshape(-1, 1),
      p["W_u2"].T, p["b_u2"].reshape(-1, 1),
      Wt.reshape(-1, 1), bt.reshape(-1, 1),
      p["W_f2"].T, p["b_f2"].reshape(-1, 1),
      p["W_f3"].T, p["b_f3"].reshape(-1, 1))
    return outT.T
